# baseline (speedup 1.0000x reference)
import functools

import jax
import jax.numpy as jnp
from jax import lax
from jax.experimental import pallas as pl
from jax.experimental.pallas import tpu as pltpu
from jax.experimental.pallas import tpu_sc as plsc

B = 16384
D = 16
NC = 2
NS = 16
NW = NC * NS
BPW = B // NW
CHUNK = 128
NCHUNK = BPW // CHUNK
G = 16

_mesh = plsc.VectorSubcoreMesh(core_axis_name="c", subcore_axis_name="s")


@functools.partial(
    pl.kernel,
    out_type=jax.ShapeDtypeStruct((NW, BPW), jnp.float32),
    mesh=_mesh,
    scratch_types=[
        pltpu.VMEM((NCHUNK, CHUNK), jnp.int32),
        pltpu.VMEM((NCHUNK, CHUNK), jnp.int32),
        pltpu.VMEM((BPW, D), jnp.float32),
        pltpu.VMEM((BPW, D), jnp.float32),
        pltpu.VMEM((BPW,), jnp.float32),
        pltpu.SemaphoreType.DMA,
    ],
    compiler_params=pltpu.CompilerParams(needs_layout_passes=False,
                                         use_tc_tiling_on_sc=False),
)
def _mf_scores(uid_hbm, iid_hbm, utab_hbm, itab_hbm, out_hbm,
               uidx_v, iidx_v, urows_v, irows_v, out_v, sem):
    wid = lax.axis_index("s") * NC + lax.axis_index("c")

    pltpu.sync_copy(uid_hbm.at[wid], uidx_v)
    pltpu.sync_copy(iid_hbm.at[wid], iidx_v)

    copies = []
    for j in range(NCHUNK):
        rows = pl.ds(j * CHUNK, CHUNK)
        copies.append(pltpu.async_copy(utab_hbm.at[uidx_v.at[j]],
                                       urows_v.at[rows], sem))
        copies.append(pltpu.async_copy(itab_hbm.at[iidx_v.at[j]],
                                       irows_v.at[rows], sem))
    for c in copies:
        c.wait()

    lanes = lax.iota(jnp.int32, G)

    def step(g, carry):
        base = g * G
        acc = jnp.zeros((G,), jnp.float32)
        for k in range(G):
            u = urows_v[base + k]
            it = irows_v[base + k]
            s = jnp.sum(u * it)
            acc = jnp.where(lanes == k, s, acc)
        rating = 1.0 + 4.0 / (1.0 + jnp.exp(-acc))
        plsc.store_scatter(out_v, [base + lanes], rating)
        return carry

    lax.fori_loop(0, BPW // G, step, 0)

    pltpu.sync_copy(out_v, out_hbm.at[wid])


def kernel(inputs, user_table, item_table):
    uid = inputs[:, 0].astype(jnp.int32).reshape(NW, NCHUNK, CHUNK)
    iid = inputs[:, 1].astype(jnp.int32).reshape(NW, NCHUNK, CHUNK)
    out = _mf_scores(uid, iid, user_table, item_table)
    return out.reshape(B)


# speedup vs baseline: 1.8153x; 1.8153x over previous
import functools

import jax
import jax.numpy as jnp
from jax import lax
from jax.experimental import pallas as pl
from jax.experimental.pallas import tpu as pltpu
from jax.experimental.pallas import tpu_sc as plsc

B = 16384
D = 16
NC = 2
NS = 16
NW = NC * NS
BPW = B // NW
G = 16

_mesh = plsc.VectorSubcoreMesh(core_axis_name="c", subcore_axis_name="s")


@functools.partial(
    pl.kernel,
    out_type=jax.ShapeDtypeStruct((NW, BPW), jnp.float32),
    mesh=_mesh,
    scratch_types=[
        pltpu.VMEM((BPW,), jnp.int32),
        pltpu.VMEM((BPW,), jnp.int32),
        pltpu.VMEM((D, 128), jnp.float32),
        pltpu.VMEM((D, 128), jnp.float32),
        pltpu.VMEM((BPW,), jnp.float32),
        pltpu.SemaphoreType.DMA,
    ],
    compiler_params=pltpu.CompilerParams(needs_layout_passes=False),
)
def _mf_scores(uid_hbm, iid_hbm, utab_hbm, itab_hbm, out_hbm,
               uidx_v, iidx_v, ublk_v, iblk_v, out_v, sem):
    wid = lax.axis_index("s") * NC + lax.axis_index("c")

    pltpu.sync_copy(uid_hbm.at[wid], uidx_v)
    pltpu.sync_copy(iid_hbm.at[wid], iidx_v)

    lanes = lax.iota(jnp.int32, G)

    def group(g, carry):
        uvec = uidx_v[pl.ds(g * G, G)]
        ivec = iidx_v[pl.ds(g * G, G)]
        acc = jnp.zeros((G,), jnp.float32)
        for j in range(G):
            r_u = jnp.sum(jnp.where(lanes == j, uvec, 0))
            r_i = jnp.sum(jnp.where(lanes == j, ivec, 0))
            blk_u = pl.multiple_of((r_u // 128) * 128, 128)
            blk_i = pl.multiple_of((r_i // 128) * 128, 128)
            cp_u = pltpu.async_copy(
                utab_hbm.at[:, pl.ds(blk_u, 128)], ublk_v, sem)
            cp_i = pltpu.async_copy(
                itab_hbm.at[:, pl.ds(blk_i, 128)], iblk_v, sem)
            cp_u.wait()
            cp_i.wait()
            lu = jnp.full((G,), r_u % 128, jnp.int32)
            li = jnp.full((G,), r_i % 128, jnp.int32)
            u = plsc.load_gather(ublk_v, [lanes, lu])
            it = plsc.load_gather(iblk_v, [lanes, li])
            s = jnp.sum(u * it)
            acc = jnp.where(lanes == j, s, acc)
        out_v[pl.ds(g * G, G)] = 1.0 + 4.0 / (1.0 + jnp.exp(-acc))
        return carry

    lax.fori_loop(0, BPW // G, group, 0)
    pltpu.sync_copy(out_v, out_hbm.at[wid])


def kernel(inputs, user_table, item_table):
    uid = inputs[:, 0].astype(jnp.int32).reshape(NW, BPW)
    iid = inputs[:, 1].astype(jnp.int32).reshape(NW, BPW)
    out = _mf_scores(uid, iid, user_table.T, item_table.T)
    return out.reshape(B)


# speedup vs baseline: 5.9549x; 3.2804x over previous
import functools

import jax
import jax.numpy as jnp
from jax import lax
from jax.experimental import pallas as pl
from jax.experimental.pallas import tpu as pltpu
from jax.experimental.pallas import tpu_sc as plsc

B = 16384
D = 16
NC = 2
NS = 16
NW = NC * NS
BPW = B // NW
G = 16
H = 8
NG = BPW // G

_mesh = plsc.VectorSubcoreMesh(core_axis_name="c", subcore_axis_name="s")


@functools.partial(
    pl.kernel,
    out_type=jax.ShapeDtypeStruct((NW, BPW), jnp.float32),
    mesh=_mesh,
    scratch_types=[
        pltpu.VMEM((BPW,), jnp.int32),
        pltpu.VMEM((BPW,), jnp.int32),
        pltpu.VMEM((2, H, D, 128), jnp.float32),
        pltpu.VMEM((2, H, D, 128), jnp.float32),
        pltpu.VMEM((BPW,), jnp.float32),
        pltpu.SemaphoreType.DMA,
        pltpu.SemaphoreType.DMA,
    ],
    compiler_params=pltpu.CompilerParams(needs_layout_passes=False),
)
def _mf_scores(uid_hbm, iid_hbm, utab_hbm, itab_hbm, out_hbm,
               uidx_v, iidx_v, ustg_v, istg_v, out_v, sem0, sem1):
    wid = lax.axis_index("s") * NC + lax.axis_index("c")

    pltpu.sync_copy(uid_hbm.at[wid], uidx_v)
    pltpu.sync_copy(iid_hbm.at[wid], iidx_v)

    lanes = lax.iota(jnp.int32, G)
    sems = (sem0, sem1)

    _dnums = lax.GatherDimensionNumbers(
        offset_dims=(), collapsed_slice_dims=(0,), start_index_map=(0,))

    def take16(vec, idx):
        return lax.gather(vec, idx[:, None], _dnums, slice_sizes=(1,),
                          mode=lax.GatherScatterMode.PROMISE_IN_BOUNDS)

    def fire(g, half, buf):
        sem = sems[buf]
        uvec = uidx_v[pl.ds(g * G, G)]
        ivec = iidx_v[pl.ds(g * G, G)]
        ublk = (uvec // 128) * 128
        iblk = (ivec // 128) * 128
        for j in range(H):
            sel = lanes == (half * H + j)
            o_u = pl.multiple_of(jnp.sum(jnp.where(sel, ublk, 0)), 128)
            o_i = pl.multiple_of(jnp.sum(jnp.where(sel, iblk, 0)), 128)
            pltpu.async_copy(utab_hbm.at[:, pl.ds(o_u, 128)],
                             ustg_v.at[buf, j], sem)
            pltpu.async_copy(itab_hbm.at[:, pl.ds(o_i, 128)],
                             istg_v.at[buf, j], sem)
        return uvec % 128, ivec % 128

    def drain(buf):
        sem = sems[buf]
        for _ in range(H):
            pltpu.make_async_copy(utab_hbm.at[:, pl.ds(0, 128)],
                                  ustg_v.at[buf, 0], sem).wait()
            pltpu.make_async_copy(itab_hbm.at[:, pl.ds(0, 128)],
                                  istg_v.at[buf, 0], sem).wait()

    def compute(half, buf, umod, imod, acc):
        for j in range(H):
            jj = jnp.full((G,), half * H + j, jnp.int32)
            lu = take16(umod, jj)
            li = take16(imod, jj)
            u = plsc.load_gather(ustg_v.at[buf, j], [lanes, lu])
            it = plsc.load_gather(istg_v.at[buf, j], [lanes, li])
            s = jnp.sum(u * it)
            acc = jnp.where(lanes == (half * H + j), s, acc)
        return acc

    dummy = lanes
    um0, im0 = fire(0, 0, 0)
    um1, im1 = fire(0, 1, 1)

    def step(g, carry):
        um0, im0, um1, im1 = carry
        acc = jnp.zeros((G,), jnp.float32)
        drain(0)
        acc = compute(0, 0, um0, im0, acc)
        nxt0 = lax.cond(g + 1 < NG,
                        lambda: fire(g + 1, 0, 0),
                        lambda: (dummy, dummy))
        drain(1)
        acc = compute(1, 1, um1, im1, acc)
        nxt1 = lax.cond(g + 1 < NG,
                        lambda: fire(g + 1, 1, 1),
                        lambda: (dummy, dummy))
        out_v[pl.ds(g * G, G)] = 1.0 + 4.0 / (1.0 + jnp.exp(-acc))
        return nxt0[0], nxt0[1], nxt1[0], nxt1[1]

    lax.fori_loop(0, NG, step, (um0, im0, um1, im1))
    pltpu.sync_copy(out_v, out_hbm.at[wid])


def kernel(inputs, user_table, item_table):
    uid = inputs[:, 0].astype(jnp.int32).reshape(NW, BPW)
    iid = inputs[:, 1].astype(jnp.int32).reshape(NW, BPW)
    out = _mf_scores(uid, iid, user_table.T, item_table.T)
    return out.reshape(B)
